# baseline (speedup 1.0000x reference)
import functools

import jax
import jax.numpy as jnp
from jax import lax
from jax.experimental import pallas as pl
from jax.experimental.pallas import tpu as pltpu
from jax.experimental.pallas import tpu_sc as plsc

D = 128
BATCH = 4096
SEQ = 200
B = BATCH * SEQ
B_SC = 729088
B_TC = B - B_SC
NROWS = 10001

NC = 2
NS = 16
NW = NC * NS
BPW = B_SC // NW
CH = 32
NCHUNK = BPW // CH
NB = 8
LOOK = 4
NIB = NB
ILOOK = NIB

_mesh = plsc.VectorSubcoreMesh(core_axis_name="c", subcore_axis_name="s")


@functools.partial(
    pl.kernel,
    mesh=_mesh,
    out_type=jax.ShapeDtypeStruct((B_SC, D), jnp.float32),
    scratch_types=(
        [pltpu.VMEM_SHARED((NROWS, D), jnp.float32)]
        + [pltpu.VMEM((CH, D), jnp.float32) for _ in range(NB)]
        + [pltpu.VMEM((CH,), jnp.int32) for _ in range(NIB)]
        + [pltpu.SemaphoreType.DMA for _ in range(2 * NB + NIB)]
    ),
)
def _gather_kernel(table_hbm, idx_hbm, out_hbm, table_sp, *scratch):
    rows = scratch[:NB]
    ibuf = scratch[NB:NB + NIB]
    gsem = scratch[NB + NIB:2 * NB + NIB]
    ssem = scratch[2 * NB + NIB:3 * NB + NIB]
    isem = scratch[3 * NB + NIB:]

    sid = lax.axis_index("s")
    wid = sid * NC + lax.axis_index("c")
    base = wid * BPW

    pltpu.sync_copy(table_hbm.at[pl.ds(sid * 624, 624)],
                    table_sp.at[pl.ds(sid * 624, 624)])

    @pl.when(sid == 0)
    def _():
        pltpu.sync_copy(table_hbm.at[pl.ds(9984, 17)],
                        table_sp.at[pl.ds(9984, 17)])

    def issue_idx(j, b):
        pltpu.async_copy(idx_hbm.at[pl.ds(base + j * CH, CH)], ibuf[b], isem[b])

    def drain_idx(b):
        pltpu.make_async_copy(idx_hbm.at[pl.ds(0, CH)], ibuf[b], isem[b]).wait()

    def issue_gather(b):
        pltpu.async_copy(table_sp.at[ibuf[b]], rows[b], gsem[b])

    def drain_gather(b):
        pltpu.make_async_copy(out_hbm.at[pl.ds(0, CH)], rows[b], gsem[b]).wait()

    def issue_scatter(j, b):
        pltpu.async_copy(rows[b], out_hbm.at[pl.ds(base + j * CH, CH)], ssem[b])

    def drain_scatter(b):
        pltpu.make_async_copy(
            rows[b], out_hbm.at[pl.ds(0, CH)], ssem[b]).wait()

    for j in range(ILOOK):
        issue_idx(j, j % NIB)
    plsc.subcore_barrier()
    for j in range(LOOK):
        drain_idx(j % NIB)
        issue_gather(j % NB)

    Nb_grp = NCHUNK // NB

    for b in range(NB):
        drain_gather(b)
        issue_scatter(b, b)
        issue_idx(b + ILOOK, b)
        b2 = (b + LOOK) % NB
        if b + LOOK >= NB:
            drain_scatter(b2)
        drain_idx(b2)
        issue_gather(b2)

    def body(g, carry):
        for b in range(NB):
            j = g * NB + b
            drain_gather(b)
            issue_scatter(j, b)
            issue_idx(j + ILOOK, b)
            b2 = (b + LOOK) % NB
            drain_scatter(b2)
            drain_idx(b2)
            issue_gather(b2)
        return carry

    lax.fori_loop(1, Nb_grp - 1, body, 0)

    for b in range(NB):
        j = (Nb_grp - 1) * NB + b
        drain_gather(b)
        issue_scatter(j, b)
        b2 = (b + LOOK) % NB
        drain_scatter(b2)
        if b < LOOK:
            drain_idx(b2)
            issue_gather(b2)

    for b in range(LOOK, NB):
        drain_scatter(b)


import math
import numpy as np

BLK = 1024
NBLK_TC = B_TC // BLK


def _divfull():
    d = np.exp(np.arange(0, D, 2, dtype=np.float32)
               * np.float32(-math.log(10000.0) / D))
    full = np.zeros((D,), dtype=np.float32)
    full[0::2] = d
    full[1::2] = d
    return full


_DIV = jnp.asarray(_divfull()).reshape(1, D)
_SCALE = np.float32(2.0 * math.pi)
_PHASE = jnp.asarray(
    np.tile(np.array([0.0, math.pi / 2.0], dtype=np.float32), D // 2)
).reshape(1, D)


def _tc_body(idx_ref, div_ref, phase_ref, out_ref):
    idx = idx_ref[...].reshape(BLK, 1)
    mask = idx > 0
    pos = (idx - 1).astype(jnp.float32)
    arg = (pos * div_ref[...]) * _SCALE + phase_ref[...]
    out_ref[...] = jnp.where(mask, jnp.sin(arg), jnp.float32(0.0))


def _tc_kernel(idx_tail):
    return pl.pallas_call(
        _tc_body,
        grid=(NBLK_TC,),
        in_specs=[
            pl.BlockSpec((1, 1, BLK), lambda i: (i, 0, 0)),
            pl.BlockSpec((1, D), lambda i: (0, 0)),
            pl.BlockSpec((1, D), lambda i: (0, 0)),
        ],
        out_specs=pl.BlockSpec((BLK, D), lambda i: (i, 0)),
        out_shape=jax.ShapeDtypeStruct((B_TC, D), jnp.float32),
    )(idx_tail.reshape(NBLK_TC, 1, BLK), _DIV, _PHASE)


def kernel(sin_table, temp_idx):
    idx = temp_idx.astype(jnp.int32).reshape(B)
    out_sc = _gather_kernel(sin_table, idx[:B_SC])
    out_tc = _tc_kernel(idx[B_SC:])
    return jnp.concatenate([out_sc, out_tc], axis=0).reshape(BATCH, SEQ, D)


# speedup vs baseline: 2.5049x; 2.5049x over previous
import functools

import jax
import jax.numpy as jnp
from jax import lax
from jax.experimental import pallas as pl
from jax.experimental.pallas import tpu as pltpu
from jax.experimental.pallas import tpu_sc as plsc

D = 128
BATCH = 4096
SEQ = 200
B = BATCH * SEQ
NROWS = 10001

NC = 2
NS = 16
NW = NC * NS
BPW = B // NW
CH = 32
NCHUNK = BPW // CH
NB = 8
LOOK = 4
NIB = NB
ILOOK = NIB

_mesh = plsc.VectorSubcoreMesh(core_axis_name="c", subcore_axis_name="s")


@functools.partial(
    pl.kernel,
    mesh=_mesh,
    out_type=jax.ShapeDtypeStruct((B, D), jnp.float32),
    scratch_types=(
        [pltpu.VMEM_SHARED((NROWS, D), jnp.float32)]
        + [pltpu.VMEM((CH, D), jnp.float32) for _ in range(NB)]
        + [pltpu.VMEM((CH,), jnp.int32) for _ in range(NIB)]
        + [pltpu.SemaphoreType.DMA for _ in range(2 * NB + NIB)]
    ),
)
def _gather_kernel(table_hbm, idx_hbm, out_hbm, table_sp, *scratch):
    rows = scratch[:NB]
    ibuf = scratch[NB:NB + NIB]
    gsem = scratch[NB + NIB:2 * NB + NIB]
    ssem = scratch[2 * NB + NIB:3 * NB + NIB]
    isem = scratch[3 * NB + NIB:]

    sid = lax.axis_index("s")
    wid = sid * NC + lax.axis_index("c")
    base = wid * BPW

    @pl.when(sid == 0)
    def _():
        pltpu.sync_copy(table_hbm, table_sp)

    def issue_idx(j, b):
        pltpu.async_copy(idx_hbm.at[pl.ds(base + j * CH, CH)], ibuf[b], isem[b])

    def drain_idx(b):
        pltpu.make_async_copy(idx_hbm.at[pl.ds(0, CH)], ibuf[b], isem[b]).wait()

    def issue_gather(b):
        pltpu.async_copy(table_sp.at[ibuf[b]], rows[b], gsem[b])

    def drain_gather(b):
        pltpu.make_async_copy(out_hbm.at[pl.ds(0, CH)], rows[b], gsem[b]).wait()

    def issue_scatter(j, b):
        pltpu.async_copy(rows[b], out_hbm.at[pl.ds(base + j * CH, CH)], ssem[b])

    def drain_scatter(b):
        pltpu.make_async_copy(
            rows[b], out_hbm.at[pl.ds(0, CH)], ssem[b]).wait()

    for j in range(ILOOK):
        issue_idx(j, j % NIB)
    plsc.subcore_barrier()
    for j in range(LOOK):
        drain_idx(j % NIB)
        issue_gather(j % NB)

    def body(g, carry):
        for b in range(NB):
            j = g * NB + b
            drain_gather(b)
            issue_scatter(j, b)

            @pl.when(j + ILOOK < NCHUNK)
            def _():
                issue_idx(j + ILOOK, b)

            jj = j + LOOK
            b2 = (b + LOOK) % NB

            @pl.when(jj >= NB)
            def _():
                drain_scatter(b2)

            @pl.when(jj < NCHUNK)
            def _():
                drain_idx((b + LOOK) % NIB)
                issue_gather(b2)
        return carry

    lax.fori_loop(0, NCHUNK // NB, body, 0)

    for b in range(LOOK, NB):
        drain_scatter(b)


def kernel(sin_table, temp_idx):
    idx = temp_idx.astype(jnp.int32).reshape(B)
    out = _gather_kernel(sin_table, idx)
    return out.reshape(BATCH, SEQ, D)
